# baseline (speedup 1.0000x reference)
import functools

import jax
import jax.numpy as jnp
from jax import lax
from jax.experimental import pallas as pl
from jax.experimental.pallas import tpu as pltpu
from jax.experimental.pallas import tpu_sc as plsc

N = 10000
D = 128
G = 16
C = 6
NC = 2
NS = 16
NW = NC * NS
CHUNK = 64
GS = 32
EPG = GS * CHUNK
NG0 = 4
NG1 = 7
ET0 = 7000
ET1 = 13000
CPT = max(NG0, NG1) * GS
NBUF = 5
N_PAD = 10008
ROWS_MAIN = 624
TAIL_OFF = (NS - 1) * ROWS_MAIN
TAIL_ROWS = N - TAIL_OFF


def _sc_agg(h, src_p, dst_p):
    mesh = plsc.VectorSubcoreMesh(core_axis_name="c", subcore_axis_name="s")

    @functools.partial(
        pl.kernel,
        out_type=[jax.ShapeDtypeStruct((N, D), jnp.float32),
                  jax.ShapeDtypeStruct((N, D), jnp.float32)],
        mesh=mesh,
        scratch_types=[
            pltpu.VMEM((GS, CHUNK), jnp.int32),
            pltpu.VMEM((GS, CHUNK), jnp.int32),
            [pltpu.VMEM((CHUNK, D), jnp.float32) for _ in range(NBUF)],
            pltpu.VMEM_SHARED((N_PAD, D), jnp.float32),
            [pltpu.SemaphoreType.DMA for _ in range(NBUF)],
        ],
    )
    def agg(h_hbm, src_hbm, dst_hbm, out0, out1,
            sg, dg, bufs, acc, sems):
        c = lax.axis_index("c")
        s = lax.axis_index("s")
        wid = s * NC + c
        row0 = pl.multiple_of(s * ROWS_MAIN, 8)

        zv = jnp.zeros((16,), jnp.float32)
        for r in range(CHUNK):
            for q in range(D // 16):
                bufs[0][r, pl.ds(q * 16, 16)] = zv

        @pl.when(s < NS - 1)
        def _zero_main():
            for t in range(ROWS_MAIN // CHUNK):
                pltpu.sync_copy(
                    bufs[0], acc.at[pl.ds(row0 + t * CHUNK, CHUNK)])
            rem = ROWS_MAIN % CHUNK
            pltpu.sync_copy(
                bufs[0].at[pl.ds(0, rem)],
                acc.at[pl.ds(row0 + (ROWS_MAIN // CHUNK) * CHUNK, rem)])

        @pl.when(s == NS - 1)
        def _zero_tail():
            for t in range(TAIL_ROWS // CHUNK):
                pltpu.sync_copy(
                    bufs[0], acc.at[pl.ds(TAIL_OFF + t * CHUNK, CHUNK)])

        plsc.subcore_barrier()

        def group(g, carry):
            g0 = pl.multiple_of(g * GS, GS)
            pltpu.sync_copy(src_hbm.at[wid, pl.ds(g0, GS)], sg)
            pltpu.sync_copy(dst_hbm.at[wid, pl.ds(g0, GS)], dg)
            for j in range(NBUF - 1):
                pltpu.async_copy(h_hbm.at[sg.at[j]], bufs[j], sems[j])
            for j in range(GS):
                k = j % NBUF
                pltpu.make_async_copy(h_hbm.at[sg.at[j]], bufs[k], sems[k]).wait()
                if j + NBUF - 1 < GS:
                    kn = (j + NBUF - 1) % NBUF
                    pltpu.async_copy(h_hbm.at[sg.at[j + NBUF - 1]],
                                     bufs[kn], sems[kn])
                pltpu.sync_copy(bufs[k], acc.at[dg.at[j]], add=True)
            return carry

        ng = lax.select(c == 0, NG0, NG1)
        lax.fori_loop(0, ng, group, 0)
        plsc.subcore_barrier()

        def copy_out(out):
            @pl.when(s < NS - 1)
            def _main():
                pltpu.sync_copy(acc.at[pl.ds(row0, ROWS_MAIN)],
                                out.at[pl.ds(row0, ROWS_MAIN)])

            @pl.when(s == NS - 1)
            def _tail():
                pltpu.sync_copy(acc.at[pl.ds(TAIL_OFF, TAIL_ROWS)],
                                out.at[pl.ds(TAIL_OFF, TAIL_ROWS)])

        @pl.when(c == 0)
        def _c0():
            copy_out(out0)

        @pl.when(c == 1)
        def _c1():
            copy_out(out1)

    return agg(h, src_p, dst_p)


_BR = 1000


def _mlp_body(h_ref, p0_ref, p1_ref, w1_ref, b1_ref, w2_ref, b2_ref, o_ref):
    z = h_ref[...] + p0_ref[...] + p1_ref[...]
    t = jnp.dot(z, w1_ref[...], preferred_element_type=jnp.float32) + b1_ref[...]
    t = jnp.maximum(t, 0.0)
    o = jnp.dot(t, w2_ref[...], preferred_element_type=jnp.float32) + b2_ref[...]
    o_ref[...] = jnp.maximum(o, 0.0)


def _mlp(h, p0, p1, W1, b1, W2, b2):
    row = pl.BlockSpec((_BR, D), lambda i: (i, 0))
    full = pl.BlockSpec((D, D), lambda i: (0, 0))
    bias = pl.BlockSpec((1, D), lambda i: (0, 0))
    return pl.pallas_call(
        _mlp_body,
        grid=(N // _BR,),
        in_specs=[row, row, row, full, bias, full, bias],
        out_specs=row,
        out_shape=jax.ShapeDtypeStruct((N, D), jnp.float32),
    )(h, p0, p1, W1, b1.reshape(1, D), W2, b2.reshape(1, D))


def _pool_heads_body(h_ref, b_ref, wp_ref, bp_ref, ws_ref, bs_ref, p_ref, s_ref):
    gids = lax.broadcasted_iota(jnp.int32, (G, N), 0)
    oh = jnp.where(b_ref[...] == gids, 1.0, 0.0)
    sums = jnp.dot(oh, h_ref[...], preferred_element_type=jnp.float32)
    counts = jnp.sum(oh, axis=1)
    pooled = sums / jnp.maximum(counts, 1.0)[:, None]
    p_ref[...] = jnp.dot(pooled, wp_ref[...], preferred_element_type=jnp.float32) + bp_ref[...]
    s_ref[...] = jnp.dot(pooled, ws_ref[...], preferred_element_type=jnp.float32) + bs_ref[...]


def _pool_heads(h, batch_row, Wp, bp, Ws, bs):
    return pl.pallas_call(
        _pool_heads_body,
        out_shape=[jax.ShapeDtypeStruct((G, C), jnp.float32),
                   jax.ShapeDtypeStruct((G, C), jnp.float32)],
    )(h, batch_row, Wp, bp.reshape(1, C), Ws, bs.reshape(1, C))


def kernel(x, edge_index, batch, W1_0, b1_0, W2_0, b2_0, W1_1, b1_1, W2_1,
           b2_1, W1_2, b1_2, W2_2, b2_2, Wp, bp, Ws, bs):
    src = edge_index[0]
    dst = edge_index[1]

    def pack(idx, fill):
        a = idx[:NS * ET0].reshape(NS, ET0)
        b = idx[NS * ET0:].reshape(NS, ET1)
        cap = CPT * CHUNK
        a = jnp.concatenate(
            [a, jnp.full((NS, cap - ET0), fill, jnp.int32)], axis=1)
        b = jnp.concatenate(
            [b, jnp.full((NS, cap - ET1), fill, jnp.int32)], axis=1)
        return jnp.stack([a, b], axis=1).reshape(NW, CPT, CHUNK)

    src_p = pack(src, 0)
    dst_p = pack(dst, N)
    batch_row = batch.reshape(1, N)
    h = x
    for (W1, b1, W2, b2) in ((W1_0, b1_0, W2_0, b2_0),
                             (W1_1, b1_1, W2_1, b2_1),
                             (W1_2, b1_2, W2_2, b2_2)):
        p0, p1 = _sc_agg(h, src_p, dst_p)
        h = _mlp(h, p0, p1, W1, b1, W2, b2)
    primary, secondary = _pool_heads(h, batch_row, Wp, bp, Ws, bs)
    return (primary, secondary)


# speedup vs baseline: 3.8331x; 3.8331x over previous
import functools

import jax
import jax.numpy as jnp
from jax import lax
from jax.experimental import pallas as pl
from jax.experimental.pallas import tpu as pltpu
from jax.experimental.pallas import tpu_sc as plsc

N = 10000
D = 128
G = 16
C = 6
NC = 2
NS = 16
NW = NC * NS
CHUNK = 64
GS = 16
EPG = GS * CHUNK
NG0 = 8
NG1 = 12
ET0 = 8000
ET1 = 12000
CPT = max(NG0, NG1) * GS
NBUF = 5
N_PAD = 10008
ROWS_MAIN = 624
TAIL_OFF = (NS - 1) * ROWS_MAIN
TAIL_ROWS = N - TAIL_OFF


def _sc_agg(h, src_p, dst_p, zeros):
    mesh = plsc.VectorSubcoreMesh(core_axis_name="c", subcore_axis_name="s")

    @functools.partial(
        pl.kernel,
        out_type=[jax.ShapeDtypeStruct((N, D), jnp.float32),
                  jax.ShapeDtypeStruct((N, D), jnp.float32)],
        mesh=mesh,
        scratch_types=[
            pltpu.VMEM((GS, CHUNK), jnp.int32),
            pltpu.VMEM((GS, CHUNK), jnp.int32),
            [pltpu.VMEM((CHUNK, D), jnp.float32) for _ in range(NBUF)],
            pltpu.VMEM_SHARED((N_PAD, D), jnp.float32),
            [pltpu.SemaphoreType.DMA for _ in range(NBUF)],
        ],
    )
    def agg(h_hbm, src_hbm, dst_hbm, z_hbm, out0, out1,
            sg, dg, bufs, acc, sems):
        c = lax.axis_index("c")
        s = lax.axis_index("s")
        wid = s * NC + c
        row0 = pl.multiple_of(s * ROWS_MAIN, 8)

        @pl.when(s < NS - 1)
        def _zero_main():
            pltpu.sync_copy(z_hbm.at[pl.ds(0, ROWS_MAIN)],
                            acc.at[pl.ds(row0, ROWS_MAIN)])

        @pl.when(s == NS - 1)
        def _zero_tail():
            pltpu.sync_copy(z_hbm, acc.at[pl.ds(TAIL_OFF, TAIL_ROWS)])

        plsc.subcore_barrier()

        def group(g, carry):
            g0 = pl.multiple_of(g * GS, GS)
            pltpu.sync_copy(src_hbm.at[wid, pl.ds(g0, GS)], sg)
            pltpu.sync_copy(dst_hbm.at[wid, pl.ds(g0, GS)], dg)
            for j in range(NBUF - 1):
                pltpu.async_copy(h_hbm.at[sg.at[j]], bufs[j], sems[j])
            for j in range(GS):
                k = j % NBUF
                pltpu.make_async_copy(h_hbm.at[sg.at[j]], bufs[k], sems[k]).wait()
                if j + NBUF - 1 < GS:
                    kn = (j + NBUF - 1) % NBUF
                    pltpu.async_copy(h_hbm.at[sg.at[j + NBUF - 1]],
                                     bufs[kn], sems[kn])
                pltpu.sync_copy(bufs[k], acc.at[dg.at[j]], add=True)
            return carry

        ng = lax.select(c == 0, NG0, NG1)
        lax.fori_loop(0, ng, group, 0)
        plsc.subcore_barrier()

        def copy_out(out):
            @pl.when(s < NS - 1)
            def _main():
                pltpu.sync_copy(acc.at[pl.ds(row0, ROWS_MAIN)],
                                out.at[pl.ds(row0, ROWS_MAIN)])

            @pl.when(s == NS - 1)
            def _tail():
                pltpu.sync_copy(acc.at[pl.ds(TAIL_OFF, TAIL_ROWS)],
                                out.at[pl.ds(TAIL_OFF, TAIL_ROWS)])

        @pl.when(c == 0)
        def _c0():
            copy_out(out0)

        @pl.when(c == 1)
        def _c1():
            copy_out(out1)

    return agg(h, src_p, dst_p, zeros)


_BR = 1000


def _mlp_body(h_ref, p0_ref, p1_ref, w1_ref, b1_ref, w2_ref, b2_ref, o_ref):
    z = h_ref[...] + p0_ref[...] + p1_ref[...]
    t = jnp.dot(z, w1_ref[...], preferred_element_type=jnp.float32) + b1_ref[...]
    t = jnp.maximum(t, 0.0)
    o = jnp.dot(t, w2_ref[...], preferred_element_type=jnp.float32) + b2_ref[...]
    o_ref[...] = jnp.maximum(o, 0.0)


def _mlp(h, p0, p1, W1, b1, W2, b2):
    row = pl.BlockSpec((_BR, D), lambda i: (i, 0))
    full = pl.BlockSpec((D, D), lambda i: (0, 0))
    bias = pl.BlockSpec((1, D), lambda i: (0, 0))
    return pl.pallas_call(
        _mlp_body,
        grid=(N // _BR,),
        in_specs=[row, row, row, full, bias, full, bias],
        out_specs=row,
        out_shape=jax.ShapeDtypeStruct((N, D), jnp.float32),
    )(h, p0, p1, W1, b1.reshape(1, D), W2, b2.reshape(1, D))


def _pool_heads_body(h_ref, b_ref, wp_ref, bp_ref, ws_ref, bs_ref, p_ref, s_ref):
    gids = lax.broadcasted_iota(jnp.int32, (G, N), 0)
    oh = jnp.where(b_ref[...] == gids, 1.0, 0.0)
    sums = jnp.dot(oh, h_ref[...], preferred_element_type=jnp.float32)
    counts = jnp.sum(oh, axis=1)
    pooled = sums / jnp.maximum(counts, 1.0)[:, None]
    p_ref[...] = jnp.dot(pooled, wp_ref[...], preferred_element_type=jnp.float32) + bp_ref[...]
    s_ref[...] = jnp.dot(pooled, ws_ref[...], preferred_element_type=jnp.float32) + bs_ref[...]


def _pool_heads(h, batch_row, Wp, bp, Ws, bs):
    return pl.pallas_call(
        _pool_heads_body,
        out_shape=[jax.ShapeDtypeStruct((G, C), jnp.float32),
                   jax.ShapeDtypeStruct((G, C), jnp.float32)],
    )(h, batch_row, Wp, bp.reshape(1, C), Ws, bs.reshape(1, C))


def kernel(x, edge_index, batch, W1_0, b1_0, W2_0, b2_0, W1_1, b1_1, W2_1,
           b2_1, W1_2, b1_2, W2_2, b2_2, Wp, bp, Ws, bs):
    src = edge_index[0]
    dst = edge_index[1]

    def pack(idx, fill):
        a = idx[:NS * ET0].reshape(NS, ET0)
        b = idx[NS * ET0:].reshape(NS, ET1)
        cap = CPT * CHUNK
        a = jnp.concatenate(
            [a, jnp.full((NS, cap - ET0), fill, jnp.int32)], axis=1)
        b = jnp.concatenate(
            [b, jnp.full((NS, cap - ET1), fill, jnp.int32)], axis=1)
        return jnp.stack([a, b], axis=1).reshape(NW, CPT, CHUNK)

    src_p = pack(src, 0)
    dst_p = pack(dst, N)
    zeros = jnp.zeros((TAIL_ROWS, D), jnp.float32)
    batch_row = batch.reshape(1, N)
    h = x
    for (W1, b1, W2, b2) in ((W1_0, b1_0, W2_0, b2_0),
                             (W1_1, b1_1, W2_1, b2_1),
                             (W1_2, b1_2, W2_2, b2_2)):
        p0, p1 = _sc_agg(h, src_p, dst_p, zeros)
        h = _mlp(h, p0, p1, W1, b1, W2, b2)
    primary, secondary = _pool_heads(h, batch_row, Wp, bp, Ws, bs)
    return (primary, secondary)


# speedup vs baseline: 3.9075x; 1.0194x over previous
import functools

import jax
import jax.numpy as jnp
from jax import lax
from jax.experimental import pallas as pl
from jax.experimental.pallas import tpu as pltpu
from jax.experimental.pallas import tpu_sc as plsc

N = 10000
D = 128
G = 16
C = 6
NC = 2
NS = 16
NW = NC * NS
CHUNK = 64
GS = 32
EPG = GS * CHUNK
NG0 = 4
NG1 = 6
ET0 = 8000
ET1 = 12000
CPT = max(NG0, NG1) * GS
NBUF = 5
N_PAD = 10008
ROWS_MAIN = 624
TAIL_OFF = (NS - 1) * ROWS_MAIN
TAIL_ROWS = N - TAIL_OFF


def _sc_agg(h, src_p, dst_p, zeros):
    mesh = plsc.VectorSubcoreMesh(core_axis_name="c", subcore_axis_name="s")

    @functools.partial(
        pl.kernel,
        out_type=[jax.ShapeDtypeStruct((N, D), jnp.float32),
                  jax.ShapeDtypeStruct((N, D), jnp.float32)],
        mesh=mesh,
        scratch_types=[
            pltpu.VMEM((GS, CHUNK), jnp.int32),
            pltpu.VMEM((GS, CHUNK), jnp.int32),
            [pltpu.VMEM((CHUNK, D), jnp.float32) for _ in range(NBUF)],
            pltpu.VMEM_SHARED((N_PAD, D), jnp.float32),
            [pltpu.SemaphoreType.DMA for _ in range(NBUF)],
        ],
    )
    def agg(h_hbm, src_hbm, dst_hbm, z_hbm, out0, out1,
            sg, dg, bufs, acc, sems):
        c = lax.axis_index("c")
        s = lax.axis_index("s")
        wid = s * NC + c
        row0 = pl.multiple_of(s * ROWS_MAIN, 8)

        @pl.when(s < NS - 1)
        def _zero_main():
            pltpu.sync_copy(z_hbm.at[pl.ds(0, ROWS_MAIN)],
                            acc.at[pl.ds(row0, ROWS_MAIN)])

        @pl.when(s == NS - 1)
        def _zero_tail():
            pltpu.sync_copy(z_hbm, acc.at[pl.ds(TAIL_OFF, TAIL_ROWS)])

        plsc.subcore_barrier()

        def group(g, carry):
            g0 = pl.multiple_of(g * GS, GS)
            pltpu.sync_copy(src_hbm.at[wid, pl.ds(g0, GS)], sg)
            pltpu.sync_copy(dst_hbm.at[wid, pl.ds(g0, GS)], dg)
            for j in range(NBUF - 1):
                pltpu.async_copy(h_hbm.at[sg.at[j]], bufs[j], sems[j])
            for j in range(GS):
                k = j % NBUF
                pltpu.make_async_copy(h_hbm.at[sg.at[j]], bufs[k], sems[k]).wait()
                if j + NBUF - 1 < GS:
                    kn = (j + NBUF - 1) % NBUF
                    pltpu.async_copy(h_hbm.at[sg.at[j + NBUF - 1]],
                                     bufs[kn], sems[kn])
                pltpu.sync_copy(bufs[k], acc.at[dg.at[j]], add=True)
            return carry

        ng = lax.select(c == 0, NG0, NG1)
        lax.fori_loop(0, ng, group, 0)
        plsc.subcore_barrier()

        def copy_out(out):
            @pl.when(s < NS - 1)
            def _main():
                pltpu.sync_copy(acc.at[pl.ds(row0, ROWS_MAIN)],
                                out.at[pl.ds(row0, ROWS_MAIN)])

            @pl.when(s == NS - 1)
            def _tail():
                pltpu.sync_copy(acc.at[pl.ds(TAIL_OFF, TAIL_ROWS)],
                                out.at[pl.ds(TAIL_OFF, TAIL_ROWS)])

        @pl.when(c == 0)
        def _c0():
            copy_out(out0)

        @pl.when(c == 1)
        def _c1():
            copy_out(out1)

    return agg(h, src_p, dst_p, zeros)


_BR = 1000


def _mlp_body(h_ref, p0_ref, p1_ref, w1_ref, b1_ref, w2_ref, b2_ref, o_ref):
    z = h_ref[...] + p0_ref[...] + p1_ref[...]
    t = jnp.dot(z, w1_ref[...], preferred_element_type=jnp.float32) + b1_ref[...]
    t = jnp.maximum(t, 0.0)
    o = jnp.dot(t, w2_ref[...], preferred_element_type=jnp.float32) + b2_ref[...]
    o_ref[...] = jnp.maximum(o, 0.0)


def _mlp(h, p0, p1, W1, b1, W2, b2):
    row = pl.BlockSpec((_BR, D), lambda i: (i, 0))
    full = pl.BlockSpec((D, D), lambda i: (0, 0))
    bias = pl.BlockSpec((1, D), lambda i: (0, 0))
    return pl.pallas_call(
        _mlp_body,
        grid=(N // _BR,),
        in_specs=[row, row, row, full, bias, full, bias],
        out_specs=row,
        out_shape=jax.ShapeDtypeStruct((N, D), jnp.float32),
    )(h, p0, p1, W1, b1.reshape(1, D), W2, b2.reshape(1, D))


def _pool_heads_body(h_ref, b_ref, wp_ref, bp_ref, ws_ref, bs_ref, p_ref, s_ref):
    gids = lax.broadcasted_iota(jnp.int32, (G, N), 0)
    oh = jnp.where(b_ref[...] == gids, 1.0, 0.0)
    sums = jnp.dot(oh, h_ref[...], preferred_element_type=jnp.float32)
    counts = jnp.sum(oh, axis=1)
    pooled = sums / jnp.maximum(counts, 1.0)[:, None]
    p_ref[...] = jnp.dot(pooled, wp_ref[...], preferred_element_type=jnp.float32) + bp_ref[...]
    s_ref[...] = jnp.dot(pooled, ws_ref[...], preferred_element_type=jnp.float32) + bs_ref[...]


def _pool_heads(h, batch_row, Wp, bp, Ws, bs):
    return pl.pallas_call(
        _pool_heads_body,
        out_shape=[jax.ShapeDtypeStruct((G, C), jnp.float32),
                   jax.ShapeDtypeStruct((G, C), jnp.float32)],
    )(h, batch_row, Wp, bp.reshape(1, C), Ws, bs.reshape(1, C))


def kernel(x, edge_index, batch, W1_0, b1_0, W2_0, b2_0, W1_1, b1_1, W2_1,
           b2_1, W1_2, b1_2, W2_2, b2_2, Wp, bp, Ws, bs):
    src = edge_index[0]
    dst = edge_index[1]

    def pack(idx, fill):
        a = idx[:NS * ET0].reshape(NS, ET0)
        b = idx[NS * ET0:].reshape(NS, ET1)
        cap = CPT * CHUNK
        a = jnp.concatenate(
            [a, jnp.full((NS, cap - ET0), fill, jnp.int32)], axis=1)
        b = jnp.concatenate(
            [b, jnp.full((NS, cap - ET1), fill, jnp.int32)], axis=1)
        return jnp.stack([a, b], axis=1).reshape(NW, CPT, CHUNK)

    src_p = pack(src, 0)
    dst_p = pack(dst, N)
    zeros = jnp.zeros((TAIL_ROWS, D), jnp.float32)
    batch_row = batch.reshape(1, N)
    h = x
    for (W1, b1, W2, b2) in ((W1_0, b1_0, W2_0, b2_0),
                             (W1_1, b1_1, W2_1, b2_1),
                             (W1_2, b1_2, W2_2, b2_2)):
        p0, p1 = _sc_agg(h, src_p, dst_p, zeros)
        h = _mlp(h, p0, p1, W1, b1, W2, b2)
    primary, secondary = _pool_heads(h, batch_row, Wp, bp, Ws, bs)
    return (primary, secondary)
